# baseline (speedup 1.0000x reference)
import functools

import jax
import jax.numpy as jnp
from jax import lax
from jax.experimental import pallas as pl
from jax.experimental.pallas import tpu as pltpu
from jax.experimental.pallas import tpu_sc as plsc

_NC = 2
_NS = 16
_NW = _NC * _NS
_L = 16

_SEQ = 32
_D = 32
_GATHER = 128
_CHUNK = 1024


def _tec_body(idx_hbm, table_hbm, pos_hbm, typ_hbm, out_hbm,
              idx_v, rows_a, rows_b, bias_v, typ_v,
              gsem_a, gsem_b, wsem_a, wsem_b):
  n_tokens = idx_hbm.shape[0] * idx_hbm.shape[1]
  rows_w = n_tokens // _NW
  grows_w = rows_w // _GATHER
  n_chunks = rows_w // _CHUNK
  calls_per_chunk = _CHUNK // _GATHER

  wid = lax.axis_index("s") * _NC + lax.axis_index("c")
  wbase = wid * rows_w

  tblr = table_hbm
  outr = out_hbm

  rows = (rows_a, rows_b)
  gsem = (gsem_a, gsem_b)
  wsem = (wsem_a, wsem_b)

  pltpu.sync_copy(idx_hbm.at[pl.ds(wid * grows_w, grows_w)], idx_v)

  def scale_idx(q, carry):
    for h in range(_GATHER // _L):
      idx_v[q, pl.ds(h * _L, _L)] = lax.shift_left(
          idx_v[q, pl.ds(h * _L, _L)], 2)
    return carry

  lax.fori_loop(0, grows_w, scale_idx, 0)

  pltpu.sync_copy(pos_hbm, bias_v)
  pltpu.sync_copy(typ_hbm, typ_v)
  for h in range(_D // _L):
    t = typ_v[0, pl.ds(h * _L, _L)]
    for r in range(_SEQ):
      bias_v[r, pl.ds(h * _L, _L)] = bias_v[r, pl.ds(h * _L, _L)] + t

  def issue_gather(c, b):
    for j in range(calls_per_chunk):
      pltpu.make_async_copy(
          tblr.at[idx_v.at[c * calls_per_chunk + j]],
          rows[b].at[pl.ds(j * _GATHER, _GATHER)],
          gsem[b]).start()

  def drain(sem):
    pltpu.make_async_copy(tblr.at[pl.ds(0, _CHUNK)], rows[0], sem).wait()

  issue_gather(0, 0)

  def chunk_body(c, carry):
    b = lax.rem(c, 2)

    def do_chunk(bi):
      ob = 1 - bi
      rbuf = rows[bi]

      @pl.when(c + 1 < n_chunks)
      def _prefetch():
        @pl.when(c >= 1)
        def _wait_wb():
          drain(wsem[ob])

        issue_gather(c + 1, ob)

      drain(gsem[bi])

      def add_bias(g, carry2):
        for r in range(_SEQ):
          for h in range(_D // _L):
            plsc.addupdate(rbuf.at[g * _SEQ + r, pl.ds(h * _L, _L)],
                           bias_v[r, pl.ds(h * _L, _L)])
        return carry2

      lax.fori_loop(0, _CHUNK // _SEQ, add_bias, 0)

      pltpu.make_async_copy(
          rbuf, outr.at[pl.ds(wbase + c * _CHUNK, _CHUNK)], wsem[bi]).start()

    @pl.when(b == 0)
    def _even():
      do_chunk(0)

    @pl.when(b == 1)
    def _odd():
      do_chunk(1)

    return carry

  lax.fori_loop(0, n_chunks, chunk_body, 0)
  drain(wsem[0])
  drain(wsem[1])


def kernel(token_ids, token_table, pos_embed, type_embed):
  batch, seq = token_ids.shape
  n_tokens = batch * seq
  idx = jnp.reshape(token_ids.astype(jnp.int32), (n_tokens // 128, 128))
  tblw = jnp.pad(token_table, ((0, 0), (0, 128 - _D)))
  tbl = jnp.reshape(tblw, (token_table.shape[0] * (128 // _D), _D))

  run = functools.partial(
      pl.kernel,
      out_type=jax.ShapeDtypeStruct((n_tokens, _D), jnp.float32),
      mesh=plsc.VectorSubcoreMesh(core_axis_name="c", subcore_axis_name="s"),
      compiler_params=pltpu.CompilerParams(use_tc_tiling_on_sc=False),
      scratch_types=[
          pltpu.VMEM((n_tokens // _NW // _GATHER, _GATHER), jnp.int32),
          pltpu.VMEM((_CHUNK, _D), jnp.float32),
          pltpu.VMEM((_CHUNK, _D), jnp.float32),
          pltpu.VMEM((_SEQ, _D), jnp.float32),
          pltpu.VMEM((1, _D), jnp.float32),
          pltpu.SemaphoreType.DMA,
          pltpu.SemaphoreType.DMA,
          pltpu.SemaphoreType.DMA,
          pltpu.SemaphoreType.DMA,
      ],
  )(_tec_body)

  out = run(idx, tbl, pos_embed, type_embed)
  return jnp.reshape(out, (batch, seq, _D))
